# baseline (speedup 1.0000x reference)
import functools

import jax
import jax.numpy as jnp
from jax.experimental import pallas as pl
from jax.experimental.pallas import tpu as pltpu

N = 8192
D = 32
K = 20
ALPHA = 3.0
BLOCK = 256
NEG = -3.4e38
INF = 3.4e38


def _proj_kernel(e1_ref, e1t_ref, e2_ref, e2t_ref, w1_ref, b1_ref,
                 w2_ref, b2_ref, c1_ref, c2_ref):
    w1t = w1_ref[...].T
    w2t = w2_ref[...].T
    t1 = jnp.tanh(ALPHA * (jnp.dot(e1_ref[...], w1t,
                                   preferred_element_type=jnp.float32)
                           + b1_ref[...][None, :]))
    t2 = jnp.tanh(ALPHA * (jnp.dot(e2_ref[...], w2t,
                                   preferred_element_type=jnp.float32)
                           + b2_ref[...][None, :]))
    t1t = jnp.tanh(ALPHA * (jnp.dot(w1_ref[...], e1t_ref[...],
                                    preferred_element_type=jnp.float32)
                            + b1_ref[...][:, None]))
    t2t = jnp.tanh(ALPHA * (jnp.dot(w2_ref[...], e2t_ref[...],
                                    preferred_element_type=jnp.float32)
                            + b2_ref[...][:, None]))
    c1_ref[:, 0:D] = t1
    c1_ref[:, D:2 * D] = t2
    c2_ref[0:D, :] = t2t
    c2_ref[D:2 * D, :] = -t1t


def _adj_kernel(c1_ref, c2_ref, out_ref):
    a = jnp.dot(c1_ref[...], c2_ref[...],
                preferred_element_type=jnp.float32)

    def _merge23(b1, c1, b2, c2):
        s1 = jnp.maximum(b1, c1)
        t1 = jnp.minimum(b1, c1)
        s2 = jnp.maximum(b2, c2)
        t2 = jnp.minimum(b2, c2)
        return s1, jnp.maximum(t1, s2), jnp.maximum(jnp.minimum(t1, s2), t2)

    def _merge33(p, q):
        b1, b2, b3 = p
        c1, c2, c3 = q
        s1 = jnp.maximum(b1, c1)
        t1 = jnp.minimum(b1, c1)
        s2 = jnp.maximum(b2, c2)
        t2 = jnp.minimum(b2, c2)
        s3 = jnp.maximum(b3, c3)
        m2 = jnp.maximum(t1, s2)
        m3 = jnp.maximum(jnp.minimum(t1, s2), jnp.maximum(t2, s3))
        return s1, m2, m3

    def _strip_top3(w):
        h = w.shape[1] // 2
        a1 = jnp.maximum(w[:, :h], w[:, h:])
        a2 = jnp.minimum(w[:, :h], w[:, h:])
        h //= 2
        a1, a2, a3 = _merge23(a1[:, :h], a1[:, h:], a2[:, :h], a2[:, h:])
        h //= 2
        while h >= 128:
            a1, a2, a3 = _merge33(
                (a1[:, :h], a2[:, :h], a3[:, :h]),
                (a1[:, h:], a2[:, h:], a3[:, h:]))
            h //= 2
        return a1, a2, a3

    STRIP = 2048
    acc = _strip_top3(a[:, :STRIP])
    for s in range(STRIP, N, STRIP):
        acc = _merge33(acc, _strip_top3(a[:, s:s + STRIP]))
    summ = jnp.concatenate(acc, axis=1)

    summ_t = summ.T
    reps = summ_t.shape[0] // 8

    def _reduce_rep(w, op):
        x = w
        while x.shape[0] > 8:
            r = x.shape[0]
            if (r // 2) % 8 == 0:
                x = op(x[:r // 2], x[r // 2:])
            else:
                x = op(op(x[:8], x[8:16]), x[16:])
        for sh in (4, 2, 1):
            x = op(x, jnp.roll(x, sh, axis=0))
        return x

    def _tree_max_rep(w):
        return _reduce_rep(w, jnp.maximum)

    def d20_body(_, t_rep):
        masked = jnp.where(summ_t < jnp.tile(t_rep, (reps, 1)), summ_t, NEG)
        return _tree_max_rep(masked)

    t_rep = jax.lax.fori_loop(
        0, K, d20_body, jnp.full((8, BLOCK), INF, jnp.float32))
    t = t_rep[0:1, :].T

    kf = float(K)
    cnt = jnp.where(summ_t >= jnp.tile(t_rep, (reps, 1)), 1.0, 0.0)
    x = _reduce_rep(cnt, jnp.add)
    dang = jnp.where(summ_t[2 * (N // 64):] >=
                     jnp.tile(t_rep, (reps // 3, 1)), 1.0, 0.0)
    d = _reduce_rep(dang, jnp.maximum)
    c = (x[0:1, :] + d[0:1, :]).T

    def raise_cond(carry):
        _t, c = carry
        return jnp.any(c > kf)

    def raise_body(carry):
        t, c = carry
        tn = jnp.min(jnp.where(a > t, a, INF), axis=1, keepdims=True)
        cn = jnp.sum(jnp.where(a >= tn, 1.0, 0.0), axis=1, keepdims=True)
        upd = jnp.logical_and(c > kf, cn >= kf)
        t = jnp.where(upd, tn, t)
        c = jnp.where(c > kf, jnp.where(cn >= kf, cn, kf), c)
        return t, c

    t, c = jax.lax.while_loop(raise_cond, raise_body, (t, c))
    out_ref[...] = jnp.where(a >= t, jnp.maximum(jnp.tanh(ALPHA * a), 0.0), 0.0)


@jax.jit
def kernel(idx, emb1_w, emb2_w, W1, b1, W2, b2):
    e1 = jnp.take(emb1_w, idx, axis=0)
    e2 = jnp.take(emb2_w, idx, axis=0)
    e1t = e1.T
    e2t = e2.T

    c1, c2 = pl.pallas_call(
        _proj_kernel,
        out_shape=(
            jax.ShapeDtypeStruct((N, 2 * D), jnp.float32),
            jax.ShapeDtypeStruct((2 * D, N), jnp.float32),
        ),
    )(e1, e1t, e2, e2t, W1, b1, W2, b2)

    grid = N // BLOCK
    out = pl.pallas_call(
        _adj_kernel,
        grid=(grid,),
        in_specs=[
            pl.BlockSpec((BLOCK, 2 * D), lambda i: (i, 0)),
            pl.BlockSpec((2 * D, N), lambda i: (0, 0)),
        ],
        out_specs=pl.BlockSpec((BLOCK, N), lambda i: (i, 0)),
        out_shape=jax.ShapeDtypeStruct((N, N), jnp.float32),
        compiler_params=pltpu.CompilerParams(
            dimension_semantics=("parallel",),
        ),
    )(c1, c2)
    return out


# speedup vs baseline: 1.0509x; 1.0509x over previous
import functools

import jax
import jax.numpy as jnp
from jax.experimental import pallas as pl
from jax.experimental.pallas import tpu as pltpu

N = 8192
D = 32
K = 20
ALPHA = 3.0
BLOCK = 256
NEG = -3.4e38
INF = 3.4e38


def _proj_kernel(e1_ref, e1t_ref, e2_ref, e2t_ref, w1_ref, b1_ref,
                 w2_ref, b2_ref, c1_ref, c2_ref):
    w1t = w1_ref[...].T
    w2t = w2_ref[...].T
    t1 = jnp.tanh(ALPHA * (jnp.dot(e1_ref[...], w1t,
                                   preferred_element_type=jnp.float32)
                           + b1_ref[...][None, :]))
    t2 = jnp.tanh(ALPHA * (jnp.dot(e2_ref[...], w2t,
                                   preferred_element_type=jnp.float32)
                           + b2_ref[...][None, :]))
    t1t = jnp.tanh(ALPHA * (jnp.dot(w1_ref[...], e1t_ref[...],
                                    preferred_element_type=jnp.float32)
                            + b1_ref[...][:, None]))
    t2t = jnp.tanh(ALPHA * (jnp.dot(w2_ref[...], e2t_ref[...],
                                    preferred_element_type=jnp.float32)
                            + b2_ref[...][:, None]))
    c1_ref[:, 0:D] = t1
    c1_ref[:, D:2 * D] = t2
    c2_ref[0:D, :] = t2t
    c2_ref[D:2 * D, :] = -t1t


def _adj_kernel(c1_ref, c2_ref, out_ref):
    a = jnp.dot(c1_ref[...], c2_ref[...],
                preferred_element_type=jnp.float32)

    def _merge23(b1, c1, b2, c2):
        s1 = jnp.maximum(b1, c1)
        t1 = jnp.minimum(b1, c1)
        s2 = jnp.maximum(b2, c2)
        t2 = jnp.minimum(b2, c2)
        return s1, jnp.maximum(t1, s2), jnp.maximum(jnp.minimum(t1, s2), t2)

    def _merge33(p, q):
        b1, b2, b3 = p
        c1, c2, c3 = q
        s1 = jnp.maximum(b1, c1)
        t1 = jnp.minimum(b1, c1)
        s2 = jnp.maximum(b2, c2)
        t2 = jnp.minimum(b2, c2)
        s3 = jnp.maximum(b3, c3)
        m2 = jnp.maximum(t1, s2)
        m3 = jnp.maximum(jnp.minimum(t1, s2), jnp.maximum(t2, s3))
        return s1, m2, m3

    def _strip_top3(w):
        h = w.shape[1] // 2
        a1 = jnp.maximum(w[:, :h], w[:, h:])
        a2 = jnp.minimum(w[:, :h], w[:, h:])
        h //= 2
        a1, a2, a3 = _merge23(a1[:, :h], a1[:, h:], a2[:, :h], a2[:, h:])
        h //= 2
        while h >= 128:
            a1, a2, a3 = _merge33(
                (a1[:, :h], a2[:, :h], a3[:, :h]),
                (a1[:, h:], a2[:, h:], a3[:, h:]))
            h //= 2
        return a1, a2, a3

    STRIP = 512
    acc = _strip_top3(a[:, :STRIP])
    for s in range(STRIP, N, STRIP):
        acc = _merge33(acc, _strip_top3(a[:, s:s + STRIP]))
    summ = jnp.concatenate(acc, axis=1)

    summ_t = summ.T
    reps = summ_t.shape[0] // 8

    def _reduce_rep(w, op):
        x = w
        while x.shape[0] > 8:
            r = x.shape[0]
            if (r // 2) % 8 == 0:
                x = op(x[:r // 2], x[r // 2:])
            else:
                x = op(op(x[:8], x[8:16]), x[16:])
        for sh in (4, 2, 1):
            x = op(x, jnp.roll(x, sh, axis=0))
        return x

    def _tree_max_rep(w):
        return _reduce_rep(w, jnp.maximum)

    def d20_body(_, t_rep):
        masked = jnp.where(summ_t < jnp.tile(t_rep, (reps, 1)), summ_t, NEG)
        return _tree_max_rep(masked)

    t_rep = jax.lax.fori_loop(
        0, K, d20_body, jnp.full((8, BLOCK), INF, jnp.float32))
    t = t_rep[0:1, :].T

    kf = float(K)
    cnt = jnp.where(summ_t >= jnp.tile(t_rep, (reps, 1)), 1.0, 0.0)
    x = _reduce_rep(cnt, jnp.add)
    dang = jnp.where(summ_t[2 * (N // 64):] >=
                     jnp.tile(t_rep, (reps // 3, 1)), 1.0, 0.0)
    d = _reduce_rep(dang, jnp.maximum)
    c = (x[0:1, :] + d[0:1, :]).T

    def raise_cond(carry):
        _t, c = carry
        return jnp.any(c > kf)

    def raise_body(carry):
        t, c = carry
        tn = jnp.min(jnp.where(a > t, a, INF), axis=1, keepdims=True)
        cn = jnp.sum(jnp.where(a >= tn, 1.0, 0.0), axis=1, keepdims=True)
        upd = jnp.logical_and(c > kf, cn >= kf)
        t = jnp.where(upd, tn, t)
        c = jnp.where(c > kf, jnp.where(cn >= kf, cn, kf), c)
        return t, c

    t, c = jax.lax.while_loop(raise_cond, raise_body, (t, c))
    out_ref[...] = jnp.where(a >= t, jnp.maximum(jnp.tanh(ALPHA * a), 0.0), 0.0)


@jax.jit
def kernel(idx, emb1_w, emb2_w, W1, b1, W2, b2):
    e1 = jnp.take(emb1_w, idx, axis=0)
    e2 = jnp.take(emb2_w, idx, axis=0)
    e1t = e1.T
    e2t = e2.T

    c1, c2 = pl.pallas_call(
        _proj_kernel,
        out_shape=(
            jax.ShapeDtypeStruct((N, 2 * D), jnp.float32),
            jax.ShapeDtypeStruct((2 * D, N), jnp.float32),
        ),
    )(e1, e1t, e2, e2t, W1, b1, W2, b2)

    grid = N // BLOCK
    out = pl.pallas_call(
        _adj_kernel,
        grid=(grid,),
        in_specs=[
            pl.BlockSpec((BLOCK, 2 * D), lambda i: (i, 0)),
            pl.BlockSpec((2 * D, N), lambda i: (0, 0)),
        ],
        out_specs=pl.BlockSpec((BLOCK, N), lambda i: (i, 0)),
        out_shape=jax.ShapeDtypeStruct((N, N), jnp.float32),
        compiler_params=pltpu.CompilerParams(
            dimension_semantics=("parallel",),
        ),
    )(c1, c2)
    return out


# speedup vs baseline: 1.1072x; 1.0536x over previous
import functools

import jax
import jax.numpy as jnp
from jax.experimental import pallas as pl
from jax.experimental.pallas import tpu as pltpu

N = 8192
D = 32
K = 20
ALPHA = 3.0
BLOCK = 256
NEG = -3.4e38
INF = 3.4e38


def _proj_kernel(e1_ref, e1t_ref, e2_ref, e2t_ref, w1_ref, b1_ref,
                 w2_ref, b2_ref, c1_ref, c2_ref):
    w1t = w1_ref[...].T
    w2t = w2_ref[...].T
    t1 = jnp.tanh(ALPHA * (jnp.dot(e1_ref[...], w1t,
                                   preferred_element_type=jnp.float32)
                           + b1_ref[...][None, :]))
    t2 = jnp.tanh(ALPHA * (jnp.dot(e2_ref[...], w2t,
                                   preferred_element_type=jnp.float32)
                           + b2_ref[...][None, :]))
    t1t = jnp.tanh(ALPHA * (jnp.dot(w1_ref[...], e1t_ref[...],
                                    preferred_element_type=jnp.float32)
                            + b1_ref[...][:, None]))
    t2t = jnp.tanh(ALPHA * (jnp.dot(w2_ref[...], e2t_ref[...],
                                    preferred_element_type=jnp.float32)
                            + b2_ref[...][:, None]))
    c1_ref[:, 0:D] = t1
    c1_ref[:, D:2 * D] = t2
    c2_ref[0:D, :] = t2t
    c2_ref[D:2 * D, :] = -t1t


def _adj_kernel(c1_ref, c2_ref, out_ref):
    a = jnp.dot(c1_ref[...], c2_ref[...],
                preferred_element_type=jnp.float32)

    def _merge23(b1, c1, b2, c2):
        s1 = jnp.maximum(b1, c1)
        t1 = jnp.minimum(b1, c1)
        s2 = jnp.maximum(b2, c2)
        t2 = jnp.minimum(b2, c2)
        return s1, jnp.maximum(t1, s2), jnp.maximum(jnp.minimum(t1, s2), t2)

    def _merge33(p, q):
        b1, b2, b3 = p
        c1, c2, c3 = q
        s1 = jnp.maximum(b1, c1)
        t1 = jnp.minimum(b1, c1)
        s2 = jnp.maximum(b2, c2)
        t2 = jnp.minimum(b2, c2)
        s3 = jnp.maximum(b3, c3)
        m2 = jnp.maximum(t1, s2)
        m3 = jnp.maximum(jnp.minimum(t1, s2), jnp.maximum(t2, s3))
        return s1, m2, m3

    def _strip_top3(w):
        h = w.shape[1] // 2
        a1 = jnp.maximum(w[:, :h], w[:, h:])
        a2 = jnp.minimum(w[:, :h], w[:, h:])
        h //= 2
        a1, a2, a3 = _merge23(a1[:, :h], a1[:, h:], a2[:, :h], a2[:, h:])
        h //= 2
        while h >= 128:
            a1, a2, a3 = _merge33(
                (a1[:, :h], a2[:, :h], a3[:, :h]),
                (a1[:, h:], a2[:, h:], a3[:, h:]))
            h //= 2
        return a1, a2, a3

    STRIP = 1024
    acc = _strip_top3(a[:, :STRIP])
    for s in range(STRIP, N, STRIP):
        acc = _merge33(acc, _strip_top3(a[:, s:s + STRIP]))
    summ = jnp.concatenate(acc, axis=1)

    summ_t = summ.T
    reps = summ_t.shape[0] // 8

    def _reduce_rep(w, op):
        x = w
        while x.shape[0] > 8:
            r = x.shape[0]
            if (r // 2) % 8 == 0:
                x = op(x[:r // 2], x[r // 2:])
            else:
                x = op(op(x[:8], x[8:16]), x[16:])
        for sh in (4, 2, 1):
            x = op(x, jnp.roll(x, sh, axis=0))
        return x

    def _tree_max_rep(w):
        return _reduce_rep(w, jnp.maximum)

    hr = summ_t.shape[0] // 2
    sa, sb = summ_t[:hr], summ_t[hr:]
    hreps = hr // 8
    ta = jnp.full((8, BLOCK), INF, jnp.float32)
    tb = ta
    tops_a, tops_b = [], []
    for _ in range(K):
        ta = _tree_max_rep(jnp.where(sa < jnp.tile(ta, (hreps, 1)), sa, NEG))
        tb = _tree_max_rep(jnp.where(sb < jnp.tile(tb, (hreps, 1)), sb, NEG))
        tops_a.append(ta)
        tops_b.append(tb)
    t_rep = jnp.maximum(tops_a[K - 1], tops_b[K - 1])
    for i in range(1, K):
        t_rep = jnp.maximum(t_rep, jnp.minimum(tops_a[i - 1], tops_b[K - 1 - i]))
    t = t_rep[0:1, :].T

    kf = float(K)
    cnt = jnp.where(summ_t >= jnp.tile(t_rep, (reps, 1)), 1.0, 0.0)
    x = _reduce_rep(cnt, jnp.add)
    dang = jnp.where(summ_t[2 * (N // 64):] >=
                     jnp.tile(t_rep, (reps // 3, 1)), 1.0, 0.0)
    d = _reduce_rep(dang, jnp.maximum)
    c = (x[0:1, :] + d[0:1, :]).T

    def raise_cond(carry):
        _t, c = carry
        return jnp.any(c > kf)

    def raise_body(carry):
        t, c = carry
        tn = jnp.min(jnp.where(a > t, a, INF), axis=1, keepdims=True)
        cn = jnp.sum(jnp.where(a >= tn, 1.0, 0.0), axis=1, keepdims=True)
        upd = jnp.logical_and(c > kf, cn >= kf)
        t = jnp.where(upd, tn, t)
        c = jnp.where(c > kf, jnp.where(cn >= kf, cn, kf), c)
        return t, c

    t, c = jax.lax.while_loop(raise_cond, raise_body, (t, c))
    out_ref[...] = jnp.where(a >= t, jnp.maximum(jnp.tanh(ALPHA * a), 0.0), 0.0)


@jax.jit
def kernel(idx, emb1_w, emb2_w, W1, b1, W2, b2):
    e1 = jnp.take(emb1_w, idx, axis=0)
    e2 = jnp.take(emb2_w, idx, axis=0)
    e1t = e1.T
    e2t = e2.T

    c1, c2 = pl.pallas_call(
        _proj_kernel,
        out_shape=(
            jax.ShapeDtypeStruct((N, 2 * D), jnp.float32),
            jax.ShapeDtypeStruct((2 * D, N), jnp.float32),
        ),
    )(e1, e1t, e2, e2t, W1, b1, W2, b2)

    grid = N // BLOCK
    out = pl.pallas_call(
        _adj_kernel,
        grid=(grid,),
        in_specs=[
            pl.BlockSpec((BLOCK, 2 * D), lambda i: (i, 0)),
            pl.BlockSpec((2 * D, N), lambda i: (0, 0)),
        ],
        out_specs=pl.BlockSpec((BLOCK, N), lambda i: (i, 0)),
        out_shape=jax.ShapeDtypeStruct((N, N), jnp.float32),
        compiler_params=pltpu.CompilerParams(
            dimension_semantics=("parallel",),
        ),
    )(c1, c2)
    return out


# speedup vs baseline: 1.1083x; 1.0009x over previous
import functools

import jax
import jax.numpy as jnp
from jax.experimental import pallas as pl
from jax.experimental.pallas import tpu as pltpu

N = 8192
D = 32
K = 20
ALPHA = 3.0
BLOCK = 256
NEG = -3.4e38
INF = 3.4e38


def _proj_kernel(e1_ref, e1t_ref, e2_ref, e2t_ref, w1_ref, b1_ref,
                 w2_ref, b2_ref, c1_ref, c2_ref):
    w1t = w1_ref[...].T
    w2t = w2_ref[...].T
    t1 = jnp.tanh(ALPHA * (jnp.dot(e1_ref[...], w1t,
                                   preferred_element_type=jnp.float32)
                           + b1_ref[...][None, :]))
    t2 = jnp.tanh(ALPHA * (jnp.dot(e2_ref[...], w2t,
                                   preferred_element_type=jnp.float32)
                           + b2_ref[...][None, :]))
    t1t = jnp.tanh(ALPHA * (jnp.dot(w1_ref[...], e1t_ref[...],
                                    preferred_element_type=jnp.float32)
                            + b1_ref[...][:, None]))
    t2t = jnp.tanh(ALPHA * (jnp.dot(w2_ref[...], e2t_ref[...],
                                    preferred_element_type=jnp.float32)
                            + b2_ref[...][:, None]))
    c1_ref[:, 0:D] = t1
    c1_ref[:, D:2 * D] = t2
    c2_ref[0:D, :] = t2t
    c2_ref[D:2 * D, :] = -t1t


def _adj_kernel(c1_ref, c2_ref, out_ref):
    a = jnp.dot(c1_ref[...], c2_ref[...],
                preferred_element_type=jnp.float32)

    def _merge23(b1, c1, b2, c2):
        s1 = jnp.maximum(b1, c1)
        t1 = jnp.minimum(b1, c1)
        s2 = jnp.maximum(b2, c2)
        t2 = jnp.minimum(b2, c2)
        return s1, jnp.maximum(t1, s2), jnp.maximum(jnp.minimum(t1, s2), t2)

    def _merge33(p, q):
        b1, b2, b3 = p
        c1, c2, c3 = q
        s1 = jnp.maximum(b1, c1)
        t1 = jnp.minimum(b1, c1)
        s2 = jnp.maximum(b2, c2)
        t2 = jnp.minimum(b2, c2)
        s3 = jnp.maximum(b3, c3)
        m2 = jnp.maximum(t1, s2)
        m3 = jnp.maximum(jnp.minimum(t1, s2), jnp.maximum(t2, s3))
        return s1, m2, m3

    def _strip_top3(w):
        h = w.shape[1] // 2
        a1 = jnp.maximum(w[:, :h], w[:, h:])
        a2 = jnp.minimum(w[:, :h], w[:, h:])
        h //= 2
        a1, a2, a3 = _merge23(a1[:, :h], a1[:, h:], a2[:, :h], a2[:, h:])
        h //= 2
        while h >= 128:
            a1, a2, a3 = _merge33(
                (a1[:, :h], a2[:, :h], a3[:, :h]),
                (a1[:, h:], a2[:, h:], a3[:, h:]))
            h //= 2
        return a1, a2, a3

    STRIP = 1024
    acc = _strip_top3(a[:, :STRIP])
    for s in range(STRIP, N, STRIP):
        acc = _merge33(acc, _strip_top3(a[:, s:s + STRIP]))
    summ = jnp.concatenate(acc, axis=1)

    summ_t = summ.T
    reps = summ_t.shape[0] // 8

    def _reduce_rep(w, op):
        x = w
        while x.shape[0] > 8:
            r = x.shape[0]
            if (r // 2) % 8 == 0:
                x = op(x[:r // 2], x[r // 2:])
            else:
                x = op(op(x[:8], x[8:16]), x[16:])
        for sh in (4, 2, 1):
            x = op(x, jnp.roll(x, sh, axis=0))
        return x

    def _tree_max_rep(w):
        return _reduce_rep(w, jnp.maximum)

    hr = summ_t.shape[0] // 2
    sa, sb = summ_t[:hr], summ_t[hr:]
    hreps = hr // 8
    ta = jnp.full((8, BLOCK), INF, jnp.float32)
    tb = ta
    tops_a, tops_b = [], []
    for _ in range(K):
        ta = _tree_max_rep(jnp.where(sa < jnp.tile(ta, (hreps, 1)), sa, NEG))
        tb = _tree_max_rep(jnp.where(sb < jnp.tile(tb, (hreps, 1)), sb, NEG))
        tops_a.append(ta)
        tops_b.append(tb)
    t_rep = jnp.maximum(tops_a[K - 1], tops_b[K - 1])
    for i in range(1, K):
        t_rep = jnp.maximum(t_rep, jnp.minimum(tops_a[i - 1], tops_b[K - 1 - i]))
    t = t_rep[0:1, :].T

    kf = float(K)
    cnt = jnp.where(summ_t >= jnp.tile(t_rep, (reps, 1)), 1.0, 0.0)
    x = _reduce_rep(cnt, jnp.add)
    dang = jnp.where(summ_t[2 * (N // 64):] >=
                     jnp.tile(t_rep, (reps // 3, 1)), 1.0, 0.0)
    d = _reduce_rep(dang, jnp.maximum)
    c = (x[0:1, :] + d[0:1, :]).T

    def raise_cond(carry):
        _t, c = carry
        return jnp.any(c > kf)

    def raise_body(carry):
        t, c = carry
        above = a > t
        tn = jnp.min(jnp.where(above, a, INF), axis=1, keepdims=True)
        cn = jnp.sum(jnp.where(above, 1.0, 0.0), axis=1, keepdims=True)
        upd = jnp.logical_and(c > kf, cn >= kf)
        t = jnp.where(upd, tn, t)
        c = jnp.where(c > kf, jnp.where(cn >= kf, cn, kf), c)
        return t, c

    t, c = jax.lax.while_loop(raise_cond, raise_body, (t, c))
    out_ref[...] = jnp.where(a >= t, jnp.maximum(jnp.tanh(ALPHA * a), 0.0), 0.0)


@jax.jit
def kernel(idx, emb1_w, emb2_w, W1, b1, W2, b2):
    e1 = jnp.take(emb1_w, idx, axis=0)
    e2 = jnp.take(emb2_w, idx, axis=0)
    e1t = e1.T
    e2t = e2.T

    c1, c2 = pl.pallas_call(
        _proj_kernel,
        out_shape=(
            jax.ShapeDtypeStruct((N, 2 * D), jnp.float32),
            jax.ShapeDtypeStruct((2 * D, N), jnp.float32),
        ),
    )(e1, e1t, e2, e2t, W1, b1, W2, b2)

    grid = N // BLOCK
    out = pl.pallas_call(
        _adj_kernel,
        grid=(grid,),
        in_specs=[
            pl.BlockSpec((BLOCK, 2 * D), lambda i: (i, 0)),
            pl.BlockSpec((2 * D, N), lambda i: (0, 0)),
        ],
        out_specs=pl.BlockSpec((BLOCK, N), lambda i: (i, 0)),
        out_shape=jax.ShapeDtypeStruct((N, N), jnp.float32),
        compiler_params=pltpu.CompilerParams(
            dimension_semantics=("parallel",),
        ),
    )(c1, c2)
    return out
